# baseline (speedup 1.0000x reference)
import functools

import jax
import jax.numpy as jnp
from jax import lax
from jax.experimental import pallas as pl
from jax.experimental.pallas import tpu as pltpu
from jax.experimental.pallas import tpu_sc as plsc

N_NODE = 10000
N_PAD = 10240
E = 320000
L = 100000
D = 128
NS = 16
G = 128

GRP = 8
E_CHUNKS_PER_TEC = GRP * -(-E // (NS * G * GRP))
E_PAD = NS * G * E_CHUNKS_PER_TEC
E_GROUPS = E_CHUNKS_PER_TEC // GRP
L_GRP = 7
L_CHUNKS_PER_TEC = -(-L // (NS * G))
L_PAD = NS * G * L_CHUNKS_PER_TEC
L_GROUPS = L_CHUNKS_PER_TEC // L_GRP

_mesh = lambda: plsc.VectorSubcoreMesh(core_axis_name="c", subcore_axis_name="s")


def _make_agg(want_deg):
    scratch = [
        pltpu.VMEM((G,), jnp.int32),
        pltpu.VMEM((G,), jnp.int32),
        pltpu.VMEM((G,), jnp.int32),
        pltpu.VMEM((G,), jnp.int32),
        pltpu.VMEM((G, D), jnp.float32),
        pltpu.VMEM_SHARED((N_PAD, D), jnp.float32),
        pltpu.SemaphoreType.DMA,
        pltpu.SemaphoreType.DMA,
        pltpu.SemaphoreType.DMA,
        pltpu.SemaphoreType.DMA,
        pltpu.SemaphoreType.DMA,
    ]
    n_out = 4 if want_deg else 2
    out_type = [jax.ShapeDtypeStruct((N_PAD, D), jnp.float32)] * n_out

    @functools.partial(pl.kernel, mesh=_mesh(), out_type=out_type,
                       scratch_types=scratch)
    def agg(ut_src, ut_dst, tu_src, tu_dst, tab_ut, tab_tu, zrows, ones,
            *rest):
        outs = rest[:n_out]
        (sidx0, sidx1, didx0, didx1, rows, accum_sh,
         gsem, ss0, ss1, ds0, ds1) = rest[n_out:]
        sidx = [sidx0, sidx1]
        didx = [didx0, didx1]
        ssem = [ss0, ss1]
        dsem = [ds0, ds1]
        c = lax.axis_index("c")
        s = lax.axis_index("s")

        rpt = N_PAD // NS
        sl = pl.ds(s * rpt, rpt)
        pltpu.sync_copy(zrows.at[sl], accum_sh.at[sl])
        plsc.subcore_barrier()

        def run(src1, dst1, tab):
            base = s * E_CHUNKS_PER_TEC * G

            def pf(j, b):
                off = base + j * G
                return (pltpu.async_copy(src1.at[pl.ds(off, G)], sidx[b],
                                         ssem[b]),
                        pltpu.async_copy(dst1.at[pl.ds(off, G)], didx[b],
                                         dsem[b]))

            def step(j, p, prefetch_next):
                if prefetch_next:
                    pf(j + 1, 1 - p)
                ssem_w = pltpu.make_async_copy(src1.at[pl.ds(0, G)],
                                               sidx[p], ssem[p])
                dsem_w = pltpu.make_async_copy(dst1.at[pl.ds(0, G)],
                                               didx[p], dsem[p])
                ssem_w.wait()
                dsem_w.wait()
                pltpu.async_copy(tab.at[sidx[p]], rows, gsem).wait()
                pltpu.sync_copy(rows, accum_sh.at[didx[p]], add=True)

            pf(0, 0)

            def pair(j2, carry):
                step(2 * j2, 0, True)
                step(2 * j2 + 1, 1, True)
                return carry

            lax.fori_loop(0, E_CHUNKS_PER_TEC // 2 - 1, pair, 0,
                          unroll=False)
            step(E_CHUNKS_PER_TEC - 2, 0, True)
            step(E_CHUNKS_PER_TEC - 1, 1, False)

        @pl.when(c == 0)
        def _():
            run(ut_src, ut_dst, tab_ut)

        @pl.when(c == 1)
        def _():
            run(tu_src, tu_dst, tab_tu)

        plsc.subcore_barrier()

        @pl.when(c == 0)
        def _():
            pltpu.sync_copy(accum_sh.at[sl], outs[0].at[sl])

        @pl.when(c == 1)
        def _():
            pltpu.sync_copy(accum_sh.at[sl], outs[1].at[sl])

        if want_deg:
            plsc.subcore_barrier()
            pltpu.sync_copy(zrows.at[sl], accum_sh.at[sl])
            pltpu.sync_copy(ones, rows)
            plsc.subcore_barrier()

            def drun(dst1):
                base = s * E_CHUNKS_PER_TEC * G

                def dpf(j, b):
                    off = base + j * G
                    return pltpu.async_copy(dst1.at[pl.ds(off, G)],
                                            didx[b], dsem[b])

                def dstep(j, p, prefetch_next):
                    if prefetch_next:
                        dpf(j + 1, 1 - p)
                    pltpu.make_async_copy(dst1.at[pl.ds(0, G)], didx[p],
                                          dsem[p]).wait()
                    pltpu.sync_copy(rows, accum_sh.at[didx[p]], add=True)

                dpf(0, 0)

                def pair(j2, carry):
                    dstep(2 * j2, 0, True)
                    dstep(2 * j2 + 1, 1, True)
                    return carry

                lax.fori_loop(0, E_CHUNKS_PER_TEC // 2 - 1, pair, 0,
                              unroll=False)
                dstep(E_CHUNKS_PER_TEC - 2, 0, True)
                dstep(E_CHUNKS_PER_TEC - 1, 1, False)

            @pl.when(c == 0)
            def _():
                drun(ut_dst)

            @pl.when(c == 1)
            def _():
                drun(tu_dst)

            plsc.subcore_barrier()

            @pl.when(c == 0)
            def _():
                pltpu.sync_copy(accum_sh.at[sl], outs[2].at[sl])

            @pl.when(c == 1)
            def _():
                pltpu.sync_copy(accum_sh.at[sl], outs[3].at[sl])

    return agg


_agg_l1 = _make_agg(True)
_agg_l2 = _make_agg(False)


def _dec_gather():
    out_type = [jax.ShapeDtypeStruct((L_PAD, D), jnp.float32),
                jax.ShapeDtypeStruct((L_PAD, D), jnp.float32)]
    scratch = [
        pltpu.VMEM((L_GRP * G,), jnp.int32),
        pltpu.VMEM((G, D), jnp.float32),
        pltpu.VMEM((G, D), jnp.float32),
        pltpu.SemaphoreType.DMA,
        pltpu.SemaphoreType.DMA,
    ]

    @functools.partial(pl.kernel, mesh=_mesh(), out_type=out_type,
                       scratch_types=scratch)
    def gat(idx_u, idx_i, z_user, z_item, out_u, out_i,
            sidx_blk, rows0, rows1, gs0, gs1):
        rows = [rows0, rows1]
        gsem = [gs0, gs1]
        c = lax.axis_index("c")
        s = lax.axis_index("s")
        base = s * L_CHUNKS_PER_TEC

        def run(idx1, tab, out):
            def group(g, carry):
                goff = (base + g * L_GRP) * G
                pltpu.sync_copy(idx1.at[pl.ds(goff, L_GRP * G)], sidx_blk)
                gcp = [None] * L_GRP
                gcp[0] = pltpu.async_copy(
                    tab.at[sidx_blk.at[pl.ds(0, G)]], rows[0], gsem[0])
                for k in range(L_GRP):
                    b = k % 2
                    if k + 1 < L_GRP:
                        nb = (k + 1) % 2
                        gcp[k + 1] = pltpu.async_copy(
                            tab.at[sidx_blk.at[pl.ds((k + 1) * G, G)]],
                            rows[nb], gsem[nb])
                    gcp[k].wait()
                    pltpu.sync_copy(rows[b], out.at[pl.ds(goff + k * G, G)])
                return carry
            lax.fori_loop(0, L_GROUPS, group, 0, unroll=False)

        @pl.when(c == 0)
        def _():
            run(idx_u, z_user, out_u)

        @pl.when(c == 1)
        def _():
            run(idx_i, z_item, out_i)

    return gat


_dec_gather_k = _dec_gather()

BLK = 1024


def _make_stage(relu):
    def body(s_ut, dg_ut, x_i, Wl_ut, bl_ut, Wr_ut,
             s_tu, dg_tu, x_u, Wl_tu, bl_tu, Wr_tu, h_i, h_u):
        def one(sref, dref, xref, Wl, bl, Wr, out):
            deg = dref[...][:, :1]
            mean = sref[...] / jnp.maximum(deg, 1.0)
            r = jnp.dot(mean, Wl[...], preferred_element_type=jnp.float32)
            r = r + bl[...] + jnp.dot(xref[...], Wr[...],
                                      preferred_element_type=jnp.float32)
            out[...] = jnp.maximum(r, 0.0) if relu else r
        one(s_ut, dg_ut, x_i, Wl_ut, bl_ut, Wr_ut, h_i)
        one(s_tu, dg_tu, x_u, Wl_tu, bl_tu, Wr_tu, h_u)

    srow = pl.BlockSpec((BLK, D), lambda i: (i, 0))
    drow = pl.BlockSpec((BLK, D), lambda i: (i, 0))
    xrow = pl.BlockSpec((BLK, D), lambda i: (i, 0))
    wb = pl.BlockSpec((D, D), lambda i: (0, 0))
    bb = pl.BlockSpec((1, D), lambda i: (0, 0))
    return pl.pallas_call(
        body,
        grid=(N_PAD // BLK,),
        in_specs=[srow, drow, xrow, wb, bb, wb, srow, drow, xrow, wb, bb, wb],
        out_specs=[xrow, xrow],
        out_shape=[jax.ShapeDtypeStruct((N_PAD, D), jnp.float32),
                   jax.ShapeDtypeStruct((N_PAD, D), jnp.float32)],
    )


_stage1 = _make_stage(True)
_stage2 = _make_stage(False)

DBLK = 1024


def _dec_mlp():
    def body(zu, zi, Wt, Wb, b1, w2, b2, out):
        a = jnp.dot(zu[...], Wt[...], preferred_element_type=jnp.float32)
        bq = jnp.dot(zi[...], Wb[...], preferred_element_type=jnp.float32)
        c = jnp.dot(zi[...], Wt[...], preferred_element_type=jnp.float32)
        d = jnp.dot(zu[...], Wb[...], preferred_element_type=jnp.float32)
        f = jnp.maximum(a + bq + b1[...], 0.0) + jnp.maximum(c + d + b1[...], 0.0)
        out[...] = 0.5 * jnp.sum(f * w2[...], axis=1) + b2[0, 0]

    row = pl.BlockSpec((DBLK, D), lambda i: (i, 0))
    wb = pl.BlockSpec((D, D), lambda i: (0, 0))
    bb = pl.BlockSpec((1, D), lambda i: (0, 0))
    sb = pl.BlockSpec(memory_space=pltpu.MemorySpace.SMEM)
    return pl.pallas_call(
        body,
        grid=(L_PAD // DBLK,),
        in_specs=[row, row, wb, wb, bb, bb, sb],
        out_specs=pl.BlockSpec((DBLK,), lambda i: (i,)),
        out_shape=jax.ShapeDtypeStruct((L_PAD,), jnp.float32),
    )


_dec_mlp_k = _dec_mlp()


def _pad_edges(ei):
    n = E_PAD - ei.shape[1]
    src = jnp.concatenate([ei[0], jnp.zeros((n,), ei.dtype)])
    dst = jnp.concatenate([ei[1], jnp.full((n,), N_NODE, ei.dtype)])
    return src, dst


def kernel(x_user, x_item, edge_index_ut, edge_index_tu, edge_label_index,
           W1l_ut, b1l_ut, W1r_ut, W1l_tu, b1l_tu, W1r_tu,
           W2l_ut, b2l_ut, W2r_ut, W2l_tu, b2l_tu, W2r_tu,
           Wd1, bd1, Wd2, bd2):
    ut_s, ut_d = _pad_edges(edge_index_ut)
    tu_s, tu_d = _pad_edges(edge_index_tu)
    zrows = jnp.zeros((N_PAD, D), jnp.float32)
    ones_blk = jnp.ones((G, D), jnp.float32)
    xpad = jnp.zeros((N_PAD - N_NODE, D), jnp.float32)
    xu_p = jnp.concatenate([x_user, xpad])
    xi_p = jnp.concatenate([x_item, xpad])

    s_ut, s_tu, deg_ut, deg_tu = _agg_l1(
        ut_s, ut_d, tu_s, tu_d, xu_p, xi_p, zrows, ones_blk)
    h_item, h_user = _stage1(
        s_ut, deg_ut, xi_p, W1l_ut, b1l_ut.reshape(1, D), W1r_ut,
        s_tu, deg_tu, xu_p, W1l_tu, b1l_tu.reshape(1, D), W1r_tu)

    s2_ut, s2_tu = _agg_l2(
        ut_s, ut_d, tu_s, tu_d, h_user, h_item, zrows, ones_blk)
    z_item, z_user = _stage2(
        s2_ut, deg_ut, h_item, W2l_ut, b2l_ut.reshape(1, D), W2r_ut,
        s2_tu, deg_tu, h_user, W2l_tu, b2l_tu.reshape(1, D), W2r_tu)

    pad = L_PAD - L
    eu1 = jnp.concatenate([edge_label_index[0],
                           jnp.zeros((pad,), edge_label_index.dtype)])
    ei1 = jnp.concatenate([edge_label_index[1],
                           jnp.zeros((pad,), edge_label_index.dtype)])
    zu_g, zi_g = _dec_gather_k(eu1, ei1, z_user, z_item)

    out = _dec_mlp_k(zu_g, zi_g, Wd1[:D], Wd1[D:], bd1.reshape(1, D),
                     Wd2.reshape(1, D), bd2.reshape(1, 1))
    return out[:L]


# speedup vs baseline: 1.2438x; 1.2438x over previous
import functools

import jax
import jax.numpy as jnp
from jax import lax
from jax.experimental import pallas as pl
from jax.experimental.pallas import tpu as pltpu
from jax.experimental.pallas import tpu_sc as plsc

N_NODE = 10000
N_PAD = 10240
E = 320000
L = 100000
D = 128
NS = 16
G = 128

E_CHUNKS_PER_TEC = -(-E // (NS * G))
E_PAD = NS * G * E_CHUNKS_PER_TEC
L_GRP = 7
L_CHUNKS_PER_TEC = -(-L // (NS * G))
L_PAD = NS * G * L_CHUNKS_PER_TEC
L_GROUPS = L_CHUNKS_PER_TEC // L_GRP

_mesh = lambda: plsc.VectorSubcoreMesh(core_axis_name="c", subcore_axis_name="s")


def _make_agg(want_deg):
    scratch = [
        pltpu.VMEM((G,), jnp.int32),
        pltpu.VMEM((G,), jnp.int32),
        pltpu.VMEM((G, D), jnp.float32),
        pltpu.VMEM_SHARED((N_PAD, D), jnp.float32),
        pltpu.SemaphoreType.DMA,
    ]
    n_out = 4 if want_deg else 2
    out_type = [jax.ShapeDtypeStruct((N_PAD, D), jnp.float32)] * n_out

    @functools.partial(pl.kernel, mesh=_mesh(), out_type=out_type,
                       scratch_types=scratch)
    def agg(ut_src, ut_dst, tu_src, tu_dst, tab_ut, tab_tu, zrows, ones,
            *rest):
        outs, (sidx, didx, rows, accum_sh, gsem) = rest[:n_out], rest[n_out:]
        c = lax.axis_index("c")
        s = lax.axis_index("s")

        rpt = N_PAD // NS
        sl = pl.ds(s * rpt, rpt)
        pltpu.sync_copy(zrows.at[sl], accum_sh.at[sl])
        plsc.subcore_barrier()

        def run(src1, dst1, tab):
            base = s * E_CHUNKS_PER_TEC * G

            def body(j, carry):
                off = base + j * G
                pltpu.sync_copy(src1.at[pl.ds(off, G)], sidx)
                pltpu.sync_copy(dst1.at[pl.ds(off, G)], didx)
                pltpu.async_copy(tab.at[sidx], rows, gsem).wait()
                pltpu.sync_copy(rows, accum_sh.at[didx], add=True)
                return carry

            lax.fori_loop(0, E_CHUNKS_PER_TEC, body, 0, unroll=False)

        @pl.when(c == 0)
        def _():
            run(ut_src, ut_dst, tab_ut)

        @pl.when(c == 1)
        def _():
            run(tu_src, tu_dst, tab_tu)

        plsc.subcore_barrier()

        @pl.when(c == 0)
        def _():
            pltpu.sync_copy(accum_sh.at[sl], outs[0].at[sl])

        @pl.when(c == 1)
        def _():
            pltpu.sync_copy(accum_sh.at[sl], outs[1].at[sl])

        if want_deg:
            plsc.subcore_barrier()
            pltpu.sync_copy(zrows.at[sl], accum_sh.at[sl])
            pltpu.sync_copy(ones, rows)
            plsc.subcore_barrier()

            def drun(dst1):
                base = s * E_CHUNKS_PER_TEC * G

                def body(j, carry):
                    off = base + j * G
                    pltpu.sync_copy(dst1.at[pl.ds(off, G)], didx)
                    pltpu.sync_copy(rows, accum_sh.at[didx], add=True)
                    return carry

                lax.fori_loop(0, E_CHUNKS_PER_TEC, body, 0, unroll=False)

            @pl.when(c == 0)
            def _():
                drun(ut_dst)

            @pl.when(c == 1)
            def _():
                drun(tu_dst)

            plsc.subcore_barrier()

            @pl.when(c == 0)
            def _():
                pltpu.sync_copy(accum_sh.at[sl], outs[2].at[sl])

            @pl.when(c == 1)
            def _():
                pltpu.sync_copy(accum_sh.at[sl], outs[3].at[sl])

    return agg


_agg_l1 = _make_agg(True)
_agg_l2 = _make_agg(False)


def _dec_gather():
    out_type = [jax.ShapeDtypeStruct((L_PAD, D), jnp.float32),
                jax.ShapeDtypeStruct((L_PAD, D), jnp.float32)]
    scratch = [
        pltpu.VMEM((L_GRP * G,), jnp.int32),
        pltpu.VMEM((G, D), jnp.float32),
        pltpu.VMEM((G, D), jnp.float32),
        pltpu.SemaphoreType.DMA,
        pltpu.SemaphoreType.DMA,
    ]

    @functools.partial(pl.kernel, mesh=_mesh(), out_type=out_type,
                       scratch_types=scratch)
    def gat(idx_u, idx_i, z_user, z_item, out_u, out_i,
            sidx_blk, rows0, rows1, gs0, gs1):
        rows = [rows0, rows1]
        gsem = [gs0, gs1]
        c = lax.axis_index("c")
        s = lax.axis_index("s")
        base = s * L_CHUNKS_PER_TEC

        def run(idx1, tab, out):
            def group(g, carry):
                goff = (base + g * L_GRP) * G
                pltpu.sync_copy(idx1.at[pl.ds(goff, L_GRP * G)], sidx_blk)
                gcp = [None] * L_GRP
                gcp[0] = pltpu.async_copy(
                    tab.at[sidx_blk.at[pl.ds(0, G)]], rows[0], gsem[0])
                for k in range(L_GRP):
                    b = k % 2
                    if k + 1 < L_GRP:
                        nb = (k + 1) % 2
                        gcp[k + 1] = pltpu.async_copy(
                            tab.at[sidx_blk.at[pl.ds((k + 1) * G, G)]],
                            rows[nb], gsem[nb])
                    gcp[k].wait()
                    pltpu.sync_copy(rows[b], out.at[pl.ds(goff + k * G, G)])
                return carry
            lax.fori_loop(0, L_GROUPS, group, 0, unroll=False)

        @pl.when(c == 0)
        def _():
            run(idx_u, z_user, out_u)

        @pl.when(c == 1)
        def _():
            run(idx_i, z_item, out_i)

    return gat


_dec_gather_k = _dec_gather()

BLK = 1024


def _make_stage(relu):
    def body(s_ut, dg_ut, x_i, Wl_ut, bl_ut, Wr_ut,
             s_tu, dg_tu, x_u, Wl_tu, bl_tu, Wr_tu, h_i, h_u):
        def one(sref, dref, xref, Wl, bl, Wr, out):
            deg = dref[...][:, :1]
            mean = sref[...] / jnp.maximum(deg, 1.0)
            r = jnp.dot(mean, Wl[...], preferred_element_type=jnp.float32)
            r = r + bl[...] + jnp.dot(xref[...], Wr[...],
                                      preferred_element_type=jnp.float32)
            out[...] = jnp.maximum(r, 0.0) if relu else r
        one(s_ut, dg_ut, x_i, Wl_ut, bl_ut, Wr_ut, h_i)
        one(s_tu, dg_tu, x_u, Wl_tu, bl_tu, Wr_tu, h_u)

    srow = pl.BlockSpec((BLK, D), lambda i: (i, 0))
    drow = pl.BlockSpec((BLK, D), lambda i: (i, 0))
    xrow = pl.BlockSpec((BLK, D), lambda i: (i, 0))
    wb = pl.BlockSpec((D, D), lambda i: (0, 0))
    bb = pl.BlockSpec((1, D), lambda i: (0, 0))
    return pl.pallas_call(
        body,
        grid=(N_PAD // BLK,),
        in_specs=[srow, drow, xrow, wb, bb, wb, srow, drow, xrow, wb, bb, wb],
        out_specs=[xrow, xrow],
        out_shape=[jax.ShapeDtypeStruct((N_PAD, D), jnp.float32),
                   jax.ShapeDtypeStruct((N_PAD, D), jnp.float32)],
    )


_stage1 = _make_stage(True)
_stage2 = _make_stage(False)

DBLK = 1024


def _dec_mlp():
    def body(zu, zi, Wt, Wb, b1, w2, b2, out):
        a = jnp.dot(zu[...], Wt[...], preferred_element_type=jnp.float32)
        bq = jnp.dot(zi[...], Wb[...], preferred_element_type=jnp.float32)
        c = jnp.dot(zi[...], Wt[...], preferred_element_type=jnp.float32)
        d = jnp.dot(zu[...], Wb[...], preferred_element_type=jnp.float32)
        f = jnp.maximum(a + bq + b1[...], 0.0) + jnp.maximum(c + d + b1[...], 0.0)
        out[...] = 0.5 * jnp.sum(f * w2[...], axis=1) + b2[0, 0]

    row = pl.BlockSpec((DBLK, D), lambda i: (i, 0))
    wb = pl.BlockSpec((D, D), lambda i: (0, 0))
    bb = pl.BlockSpec((1, D), lambda i: (0, 0))
    sb = pl.BlockSpec(memory_space=pltpu.MemorySpace.SMEM)
    return pl.pallas_call(
        body,
        grid=(L_PAD // DBLK,),
        in_specs=[row, row, wb, wb, bb, bb, sb],
        out_specs=pl.BlockSpec((DBLK,), lambda i: (i,)),
        out_shape=jax.ShapeDtypeStruct((L_PAD,), jnp.float32),
    )


_dec_mlp_k = _dec_mlp()


def _pad_edges(ei):
    n = E_PAD - ei.shape[1]
    src = jnp.concatenate([ei[0], jnp.zeros((n,), ei.dtype)])
    dst = jnp.concatenate([ei[1], jnp.full((n,), N_NODE, ei.dtype)])
    return src, dst


def kernel(x_user, x_item, edge_index_ut, edge_index_tu, edge_label_index,
           W1l_ut, b1l_ut, W1r_ut, W1l_tu, b1l_tu, W1r_tu,
           W2l_ut, b2l_ut, W2r_ut, W2l_tu, b2l_tu, W2r_tu,
           Wd1, bd1, Wd2, bd2):
    ut_s, ut_d = _pad_edges(edge_index_ut)
    tu_s, tu_d = _pad_edges(edge_index_tu)
    zrows = jnp.zeros((N_PAD, D), jnp.float32)
    ones_blk = jnp.ones((G, D), jnp.float32)
    xpad = jnp.zeros((N_PAD - N_NODE, D), jnp.float32)
    xu_p = jnp.concatenate([x_user, xpad])
    xi_p = jnp.concatenate([x_item, xpad])

    s_ut, s_tu, deg_ut, deg_tu = _agg_l1(
        ut_s, ut_d, tu_s, tu_d, xu_p, xi_p, zrows, ones_blk)
    h_item, h_user = _stage1(
        s_ut, deg_ut, xi_p, W1l_ut, b1l_ut.reshape(1, D), W1r_ut,
        s_tu, deg_tu, xu_p, W1l_tu, b1l_tu.reshape(1, D), W1r_tu)

    s2_ut, s2_tu = _agg_l2(
        ut_s, ut_d, tu_s, tu_d, h_user, h_item, zrows, ones_blk)
    z_item, z_user = _stage2(
        s2_ut, deg_ut, h_item, W2l_ut, b2l_ut.reshape(1, D), W2r_ut,
        s2_tu, deg_tu, h_user, W2l_tu, b2l_tu.reshape(1, D), W2r_tu)

    pad = L_PAD - L
    eu1 = jnp.concatenate([edge_label_index[0],
                           jnp.zeros((pad,), edge_label_index.dtype)])
    ei1 = jnp.concatenate([edge_label_index[1],
                           jnp.zeros((pad,), edge_label_index.dtype)])
    zu_g, zi_g = _dec_gather_k(eu1, ei1, z_user, z_item)

    out = _dec_mlp_k(zu_g, zi_g, Wd1[:D], Wd1[D:], bd1.reshape(1, D),
                     Wd2.reshape(1, D), bd2.reshape(1, 1))
    return out[:L]


# speedup vs baseline: 1.3005x; 1.0456x over previous
import functools

import jax
import jax.numpy as jnp
from jax import lax
from jax.experimental import pallas as pl
from jax.experimental.pallas import tpu as pltpu
from jax.experimental.pallas import tpu_sc as plsc

N_NODE = 10000
N_PAD = 10240
E = 320000
L = 100000
D = 128
NS = 16
G = 128

E_CHUNKS_PER_TEC = -(-E // (NS * G))
E_PAD = NS * G * E_CHUNKS_PER_TEC
L_GRP = 7
L_CHUNKS_PER_TEC = -(-L // (NS * G))
L_PAD = NS * G * L_CHUNKS_PER_TEC
L_GROUPS = L_CHUNKS_PER_TEC // L_GRP

_mesh = lambda: plsc.VectorSubcoreMesh(core_axis_name="c", subcore_axis_name="s")


def _make_agg(want_deg):
    scratch = [
        pltpu.VMEM((G,), jnp.int32),
        pltpu.VMEM((G,), jnp.int32),
        pltpu.VMEM((G,), jnp.int32),
        pltpu.VMEM((G,), jnp.int32),
        pltpu.VMEM((G, D), jnp.float32),
        pltpu.VMEM((G, D), jnp.float32),
        pltpu.VMEM_SHARED((N_PAD, D), jnp.float32),
        pltpu.SemaphoreType.DMA,
        pltpu.SemaphoreType.DMA,
    ]
    n_out = 4 if want_deg else 2
    out_type = [jax.ShapeDtypeStruct((N_PAD, D), jnp.float32)] * n_out

    @functools.partial(pl.kernel, mesh=_mesh(), out_type=out_type,
                       scratch_types=scratch)
    def agg(ut_src, ut_dst, tu_src, tu_dst, tab_ut, tab_tu, zrows, ones,
            *rest):
        outs = rest[:n_out]
        (sidx, didx, sidxb, didxb, rows, rowsb, accum_sh,
         gsem, gsemb) = rest[n_out:]
        c = lax.axis_index("c")
        s = lax.axis_index("s")

        rpt = N_PAD // NS
        sl = pl.ds(s * rpt, rpt)
        pltpu.sync_copy(zrows.at[sl], accum_sh.at[sl])
        plsc.subcore_barrier()

        def run(src1, dst1, tab):
            base = s * E_CHUNKS_PER_TEC * G

            def pairbody(j2, carry):
                off = base + 2 * j2 * G
                pltpu.sync_copy(src1.at[pl.ds(off, G)], sidx)
                pltpu.sync_copy(dst1.at[pl.ds(off, G)], didx)
                pltpu.sync_copy(src1.at[pl.ds(off + G, G)], sidxb)
                pltpu.sync_copy(dst1.at[pl.ds(off + G, G)], didxb)
                g0 = pltpu.async_copy(tab.at[sidx], rows, gsem)
                g1 = pltpu.async_copy(tab.at[sidxb], rowsb, gsemb)
                g0.wait()
                pltpu.sync_copy(rows, accum_sh.at[didx], add=True)
                g1.wait()
                pltpu.sync_copy(rowsb, accum_sh.at[didxb], add=True)
                return carry

            lax.fori_loop(0, E_CHUNKS_PER_TEC // 2, pairbody, 0,
                          unroll=False)
            off = base + (E_CHUNKS_PER_TEC - 1) * G
            pltpu.sync_copy(src1.at[pl.ds(off, G)], sidx)
            pltpu.sync_copy(dst1.at[pl.ds(off, G)], didx)
            pltpu.async_copy(tab.at[sidx], rows, gsem).wait()
            pltpu.sync_copy(rows, accum_sh.at[didx], add=True)

        @pl.when(c == 0)
        def _():
            run(ut_src, ut_dst, tab_ut)

        @pl.when(c == 1)
        def _():
            run(tu_src, tu_dst, tab_tu)

        plsc.subcore_barrier()

        @pl.when(c == 0)
        def _():
            pltpu.sync_copy(accum_sh.at[sl], outs[0].at[sl])

        @pl.when(c == 1)
        def _():
            pltpu.sync_copy(accum_sh.at[sl], outs[1].at[sl])

        if want_deg:
            plsc.subcore_barrier()
            pltpu.sync_copy(zrows.at[sl], accum_sh.at[sl])
            pltpu.sync_copy(ones, rows)
            plsc.subcore_barrier()

            def drun(dst1):
                base = s * E_CHUNKS_PER_TEC * G

                def body(j, carry):
                    off = base + j * G
                    pltpu.sync_copy(dst1.at[pl.ds(off, G)], didx)
                    pltpu.sync_copy(rows, accum_sh.at[didx], add=True)
                    return carry

                lax.fori_loop(0, E_CHUNKS_PER_TEC, body, 0, unroll=False)

            @pl.when(c == 0)
            def _():
                drun(ut_dst)

            @pl.when(c == 1)
            def _():
                drun(tu_dst)

            plsc.subcore_barrier()

            @pl.when(c == 0)
            def _():
                pltpu.sync_copy(accum_sh.at[sl], outs[2].at[sl])

            @pl.when(c == 1)
            def _():
                pltpu.sync_copy(accum_sh.at[sl], outs[3].at[sl])

    return agg


_agg_l1 = _make_agg(True)
_agg_l2 = _make_agg(False)


def _dec_gather():
    out_type = [jax.ShapeDtypeStruct((L_PAD, D), jnp.float32),
                jax.ShapeDtypeStruct((L_PAD, D), jnp.float32)]
    scratch = [
        pltpu.VMEM((L_GRP * G,), jnp.int32),
        pltpu.VMEM((G, D), jnp.float32),
        pltpu.VMEM((G, D), jnp.float32),
        pltpu.SemaphoreType.DMA,
        pltpu.SemaphoreType.DMA,
    ]

    @functools.partial(pl.kernel, mesh=_mesh(), out_type=out_type,
                       scratch_types=scratch)
    def gat(idx_u, idx_i, z_user, z_item, out_u, out_i,
            sidx_blk, rows0, rows1, gs0, gs1):
        rows = [rows0, rows1]
        gsem = [gs0, gs1]
        c = lax.axis_index("c")
        s = lax.axis_index("s")
        base = s * L_CHUNKS_PER_TEC

        def run(idx1, tab, out):
            def group(g, carry):
                goff = (base + g * L_GRP) * G
                pltpu.sync_copy(idx1.at[pl.ds(goff, L_GRP * G)], sidx_blk)
                gcp = [None] * L_GRP
                gcp[0] = pltpu.async_copy(
                    tab.at[sidx_blk.at[pl.ds(0, G)]], rows[0], gsem[0])
                for k in range(L_GRP):
                    b = k % 2
                    if k + 1 < L_GRP:
                        nb = (k + 1) % 2
                        gcp[k + 1] = pltpu.async_copy(
                            tab.at[sidx_blk.at[pl.ds((k + 1) * G, G)]],
                            rows[nb], gsem[nb])
                    gcp[k].wait()
                    pltpu.sync_copy(rows[b], out.at[pl.ds(goff + k * G, G)])
                return carry
            lax.fori_loop(0, L_GROUPS, group, 0, unroll=False)

        @pl.when(c == 0)
        def _():
            run(idx_u, z_user, out_u)

        @pl.when(c == 1)
        def _():
            run(idx_i, z_item, out_i)

    return gat


_dec_gather_k = _dec_gather()

BLK = 1024


def _make_stage(relu):
    def body(s_ut, dg_ut, x_i, Wl_ut, bl_ut, Wr_ut,
             s_tu, dg_tu, x_u, Wl_tu, bl_tu, Wr_tu, h_i, h_u):
        def one(sref, dref, xref, Wl, bl, Wr, out):
            deg = dref[...][:, :1]
            mean = sref[...] / jnp.maximum(deg, 1.0)
            r = jnp.dot(mean, Wl[...], preferred_element_type=jnp.float32)
            r = r + bl[...] + jnp.dot(xref[...], Wr[...],
                                      preferred_element_type=jnp.float32)
            out[...] = jnp.maximum(r, 0.0) if relu else r
        one(s_ut, dg_ut, x_i, Wl_ut, bl_ut, Wr_ut, h_i)
        one(s_tu, dg_tu, x_u, Wl_tu, bl_tu, Wr_tu, h_u)

    srow = pl.BlockSpec((BLK, D), lambda i: (i, 0))
    drow = pl.BlockSpec((BLK, D), lambda i: (i, 0))
    xrow = pl.BlockSpec((BLK, D), lambda i: (i, 0))
    wb = pl.BlockSpec((D, D), lambda i: (0, 0))
    bb = pl.BlockSpec((1, D), lambda i: (0, 0))
    return pl.pallas_call(
        body,
        grid=(N_PAD // BLK,),
        in_specs=[srow, drow, xrow, wb, bb, wb, srow, drow, xrow, wb, bb, wb],
        out_specs=[xrow, xrow],
        out_shape=[jax.ShapeDtypeStruct((N_PAD, D), jnp.float32),
                   jax.ShapeDtypeStruct((N_PAD, D), jnp.float32)],
    )


_stage1 = _make_stage(True)
_stage2 = _make_stage(False)

DBLK = 1024


def _dec_mlp():
    def body(zu, zi, Wt, Wb, b1, w2, b2, out):
        a = jnp.dot(zu[...], Wt[...], preferred_element_type=jnp.float32)
        bq = jnp.dot(zi[...], Wb[...], preferred_element_type=jnp.float32)
        c = jnp.dot(zi[...], Wt[...], preferred_element_type=jnp.float32)
        d = jnp.dot(zu[...], Wb[...], preferred_element_type=jnp.float32)
        f = jnp.maximum(a + bq + b1[...], 0.0) + jnp.maximum(c + d + b1[...], 0.0)
        out[...] = 0.5 * jnp.sum(f * w2[...], axis=1) + b2[0, 0]

    row = pl.BlockSpec((DBLK, D), lambda i: (i, 0))
    wb = pl.BlockSpec((D, D), lambda i: (0, 0))
    bb = pl.BlockSpec((1, D), lambda i: (0, 0))
    sb = pl.BlockSpec(memory_space=pltpu.MemorySpace.SMEM)
    return pl.pallas_call(
        body,
        grid=(L_PAD // DBLK,),
        in_specs=[row, row, wb, wb, bb, bb, sb],
        out_specs=pl.BlockSpec((DBLK,), lambda i: (i,)),
        out_shape=jax.ShapeDtypeStruct((L_PAD,), jnp.float32),
    )


_dec_mlp_k = _dec_mlp()


def _pad_edges(ei):
    n = E_PAD - ei.shape[1]
    src = jnp.concatenate([ei[0], jnp.zeros((n,), ei.dtype)])
    dst = jnp.concatenate([ei[1], jnp.full((n,), N_NODE, ei.dtype)])
    return src, dst


def kernel(x_user, x_item, edge_index_ut, edge_index_tu, edge_label_index,
           W1l_ut, b1l_ut, W1r_ut, W1l_tu, b1l_tu, W1r_tu,
           W2l_ut, b2l_ut, W2r_ut, W2l_tu, b2l_tu, W2r_tu,
           Wd1, bd1, Wd2, bd2):
    ut_s, ut_d = _pad_edges(edge_index_ut)
    tu_s, tu_d = _pad_edges(edge_index_tu)
    zrows = jnp.zeros((N_PAD, D), jnp.float32)
    ones_blk = jnp.ones((G, D), jnp.float32)
    xpad = jnp.zeros((N_PAD - N_NODE, D), jnp.float32)
    xu_p = jnp.concatenate([x_user, xpad])
    xi_p = jnp.concatenate([x_item, xpad])

    s_ut, s_tu, deg_ut, deg_tu = _agg_l1(
        ut_s, ut_d, tu_s, tu_d, xu_p, xi_p, zrows, ones_blk)
    h_item, h_user = _stage1(
        s_ut, deg_ut, xi_p, W1l_ut, b1l_ut.reshape(1, D), W1r_ut,
        s_tu, deg_tu, xu_p, W1l_tu, b1l_tu.reshape(1, D), W1r_tu)

    s2_ut, s2_tu = _agg_l2(
        ut_s, ut_d, tu_s, tu_d, h_user, h_item, zrows, ones_blk)
    z_item, z_user = _stage2(
        s2_ut, deg_ut, h_item, W2l_ut, b2l_ut.reshape(1, D), W2r_ut,
        s2_tu, deg_tu, h_user, W2l_tu, b2l_tu.reshape(1, D), W2r_tu)

    pad = L_PAD - L
    eu1 = jnp.concatenate([edge_label_index[0],
                           jnp.zeros((pad,), edge_label_index.dtype)])
    ei1 = jnp.concatenate([edge_label_index[1],
                           jnp.zeros((pad,), edge_label_index.dtype)])
    zu_g, zi_g = _dec_gather_k(eu1, ei1, z_user, z_item)

    out = _dec_mlp_k(zu_g, zi_g, Wd1[:D], Wd1[D:], bd1.reshape(1, D),
                     Wd2.reshape(1, D), bd2.reshape(1, 1))
    return out[:L]


# speedup vs baseline: 1.3493x; 1.0376x over previous
import functools

import jax
import jax.numpy as jnp
from jax import lax
from jax.experimental import pallas as pl
from jax.experimental.pallas import tpu as pltpu
from jax.experimental.pallas import tpu_sc as plsc

N_NODE = 10000
N_PAD = 10240
E = 320000
L = 100000
D = 128
NS = 16
G = 128

E_CHUNKS_PER_TEC = -(-E // (NS * G))
E_PAD = NS * G * E_CHUNKS_PER_TEC
L_GRP = 7
L_CHUNKS_PER_TEC = -(-L // (NS * G))
L_PAD = NS * G * L_CHUNKS_PER_TEC
L_GROUPS = L_CHUNKS_PER_TEC // L_GRP

_mesh = lambda: plsc.VectorSubcoreMesh(core_axis_name="c", subcore_axis_name="s")


def _make_agg(want_deg):
    scratch = [
        pltpu.VMEM((2 * G,), jnp.int32),
        pltpu.VMEM((G,), jnp.int32),
        pltpu.VMEM((G,), jnp.int32),
        pltpu.VMEM((G, D), jnp.float32),
        pltpu.VMEM((G, D), jnp.float32),
        pltpu.VMEM_SHARED((N_PAD, D), jnp.float32),
        pltpu.SemaphoreType.DMA,
        pltpu.SemaphoreType.DMA,
    ]
    n_out = 4 if want_deg else 2
    out_type = [jax.ShapeDtypeStruct((N_PAD, D), jnp.float32)] * n_out

    @functools.partial(pl.kernel, mesh=_mesh(), out_type=out_type,
                       scratch_types=scratch)
    def agg(ut_src, ut_dst, tu_src, tu_dst, tab_ut, tab_tu, zrows, ones,
            *rest):
        outs = rest[:n_out]
        (sidx2, didx, didxb, rows, rowsb, accum_sh,
         gsem, gsemb) = rest[n_out:]
        c = lax.axis_index("c")
        s = lax.axis_index("s")

        rpt = N_PAD // NS
        sl = pl.ds(s * rpt, rpt)
        pltpu.sync_copy(zrows.at[sl], accum_sh.at[sl])
        plsc.subcore_barrier()

        def run(src1, dst1, tab):
            base = s * E_CHUNKS_PER_TEC * G

            def pairbody(j2, carry):
                off = base + 2 * j2 * G
                pltpu.sync_copy(src1.at[pl.ds(off, 2 * G)], sidx2)
                pltpu.sync_copy(dst1.at[pl.ds(off, G)], didx)
                pltpu.sync_copy(dst1.at[pl.ds(off + G, G)], didxb)
                g0 = pltpu.async_copy(tab.at[sidx2.at[pl.ds(0, G)]],
                                      rows, gsem)
                g1 = pltpu.async_copy(tab.at[sidx2.at[pl.ds(G, G)]],
                                      rowsb, gsemb)
                g0.wait()
                pltpu.sync_copy(rows, accum_sh.at[didx], add=True)
                g1.wait()
                pltpu.sync_copy(rowsb, accum_sh.at[didxb], add=True)
                return carry

            lax.fori_loop(0, E_CHUNKS_PER_TEC // 2, pairbody, 0,
                          unroll=False)
            off = base + (E_CHUNKS_PER_TEC - 1) * G
            pltpu.sync_copy(src1.at[pl.ds(off, G)],
                            sidx2.at[pl.ds(0, G)])
            pltpu.sync_copy(dst1.at[pl.ds(off, G)], didx)
            pltpu.async_copy(tab.at[sidx2.at[pl.ds(0, G)]],
                             rows, gsem).wait()
            pltpu.sync_copy(rows, accum_sh.at[didx], add=True)

        @pl.when(c == 0)
        def _():
            run(ut_src, ut_dst, tab_ut)

        @pl.when(c == 1)
        def _():
            run(tu_src, tu_dst, tab_tu)

        plsc.subcore_barrier()

        @pl.when(c == 0)
        def _():
            pltpu.sync_copy(accum_sh.at[sl], outs[0].at[sl])

        @pl.when(c == 1)
        def _():
            pltpu.sync_copy(accum_sh.at[sl], outs[1].at[sl])

        if want_deg:
            plsc.subcore_barrier()
            pltpu.sync_copy(zrows.at[sl], accum_sh.at[sl])
            pltpu.sync_copy(ones, rows)
            plsc.subcore_barrier()

            def drun(dst1):
                base = s * E_CHUNKS_PER_TEC * G

                def body(j, carry):
                    off = base + j * G
                    pltpu.sync_copy(dst1.at[pl.ds(off, G)], didx)
                    pltpu.sync_copy(rows, accum_sh.at[didx], add=True)
                    return carry

                lax.fori_loop(0, E_CHUNKS_PER_TEC, body, 0, unroll=False)

            @pl.when(c == 0)
            def _():
                drun(ut_dst)

            @pl.when(c == 1)
            def _():
                drun(tu_dst)

            plsc.subcore_barrier()

            @pl.when(c == 0)
            def _():
                pltpu.sync_copy(accum_sh.at[sl], outs[2].at[sl])

            @pl.when(c == 1)
            def _():
                pltpu.sync_copy(accum_sh.at[sl], outs[3].at[sl])

    return agg


_agg_l1 = _make_agg(True)
_agg_l2 = _make_agg(False)


def _dec_gather():
    out_type = [jax.ShapeDtypeStruct((L_PAD, D), jnp.float32),
                jax.ShapeDtypeStruct((L_PAD, D), jnp.float32)]
    scratch = [
        pltpu.VMEM((L_GRP * G,), jnp.int32),
        pltpu.VMEM((G, D), jnp.float32),
        pltpu.VMEM((G, D), jnp.float32),
        pltpu.SemaphoreType.DMA,
        pltpu.SemaphoreType.DMA,
    ]

    @functools.partial(pl.kernel, mesh=_mesh(), out_type=out_type,
                       scratch_types=scratch)
    def gat(idx_u, idx_i, z_user, z_item, out_u, out_i,
            sidx_blk, rows0, rows1, gs0, gs1):
        rows = [rows0, rows1]
        gsem = [gs0, gs1]
        c = lax.axis_index("c")
        s = lax.axis_index("s")
        base = s * L_CHUNKS_PER_TEC

        def run(idx1, tab, out):
            def group(g, carry):
                goff = (base + g * L_GRP) * G
                pltpu.sync_copy(idx1.at[pl.ds(goff, L_GRP * G)], sidx_blk)
                gcp = [None] * L_GRP
                gcp[0] = pltpu.async_copy(
                    tab.at[sidx_blk.at[pl.ds(0, G)]], rows[0], gsem[0])
                for k in range(L_GRP):
                    b = k % 2
                    if k + 1 < L_GRP:
                        nb = (k + 1) % 2
                        gcp[k + 1] = pltpu.async_copy(
                            tab.at[sidx_blk.at[pl.ds((k + 1) * G, G)]],
                            rows[nb], gsem[nb])
                    gcp[k].wait()
                    pltpu.sync_copy(rows[b], out.at[pl.ds(goff + k * G, G)])
                return carry
            lax.fori_loop(0, L_GROUPS, group, 0, unroll=False)

        @pl.when(c == 0)
        def _():
            run(idx_u, z_user, out_u)

        @pl.when(c == 1)
        def _():
            run(idx_i, z_item, out_i)

    return gat


_dec_gather_k = _dec_gather()

BLK = 1024


def _make_stage(relu):
    def body(s_ut, dg_ut, x_i, Wl_ut, bl_ut, Wr_ut,
             s_tu, dg_tu, x_u, Wl_tu, bl_tu, Wr_tu, h_i, h_u):
        def one(sref, dref, xref, Wl, bl, Wr, out):
            deg = dref[...][:, :1]
            mean = sref[...] / jnp.maximum(deg, 1.0)
            r = jnp.dot(mean, Wl[...], preferred_element_type=jnp.float32)
            r = r + bl[...] + jnp.dot(xref[...], Wr[...],
                                      preferred_element_type=jnp.float32)
            out[...] = jnp.maximum(r, 0.0) if relu else r
        one(s_ut, dg_ut, x_i, Wl_ut, bl_ut, Wr_ut, h_i)
        one(s_tu, dg_tu, x_u, Wl_tu, bl_tu, Wr_tu, h_u)

    srow = pl.BlockSpec((BLK, D), lambda i: (i, 0))
    drow = pl.BlockSpec((BLK, D), lambda i: (i, 0))
    xrow = pl.BlockSpec((BLK, D), lambda i: (i, 0))
    wb = pl.BlockSpec((D, D), lambda i: (0, 0))
    bb = pl.BlockSpec((1, D), lambda i: (0, 0))
    return pl.pallas_call(
        body,
        grid=(N_PAD // BLK,),
        in_specs=[srow, drow, xrow, wb, bb, wb, srow, drow, xrow, wb, bb, wb],
        out_specs=[xrow, xrow],
        out_shape=[jax.ShapeDtypeStruct((N_PAD, D), jnp.float32),
                   jax.ShapeDtypeStruct((N_PAD, D), jnp.float32)],
    )


_stage1 = _make_stage(True)
_stage2 = _make_stage(False)

DBLK = 1024


def _dec_mlp():
    def body(zu, zi, Wt, Wb, b1, w2, b2, out):
        a = jnp.dot(zu[...], Wt[...], preferred_element_type=jnp.float32)
        bq = jnp.dot(zi[...], Wb[...], preferred_element_type=jnp.float32)
        c = jnp.dot(zi[...], Wt[...], preferred_element_type=jnp.float32)
        d = jnp.dot(zu[...], Wb[...], preferred_element_type=jnp.float32)
        f = jnp.maximum(a + bq + b1[...], 0.0) + jnp.maximum(c + d + b1[...], 0.0)
        out[...] = 0.5 * jnp.sum(f * w2[...], axis=1) + b2[0, 0]

    row = pl.BlockSpec((DBLK, D), lambda i: (i, 0))
    wb = pl.BlockSpec((D, D), lambda i: (0, 0))
    bb = pl.BlockSpec((1, D), lambda i: (0, 0))
    sb = pl.BlockSpec(memory_space=pltpu.MemorySpace.SMEM)
    return pl.pallas_call(
        body,
        grid=(L_PAD // DBLK,),
        in_specs=[row, row, wb, wb, bb, bb, sb],
        out_specs=pl.BlockSpec((DBLK,), lambda i: (i,)),
        out_shape=jax.ShapeDtypeStruct((L_PAD,), jnp.float32),
    )


_dec_mlp_k = _dec_mlp()


def _pad_edges(ei):
    n = E_PAD - ei.shape[1]
    src = jnp.concatenate([ei[0], jnp.zeros((n,), ei.dtype)])
    dst = jnp.concatenate([ei[1], jnp.full((n,), N_NODE, ei.dtype)])
    return src, dst


def kernel(x_user, x_item, edge_index_ut, edge_index_tu, edge_label_index,
           W1l_ut, b1l_ut, W1r_ut, W1l_tu, b1l_tu, W1r_tu,
           W2l_ut, b2l_ut, W2r_ut, W2l_tu, b2l_tu, W2r_tu,
           Wd1, bd1, Wd2, bd2):
    ut_s, ut_d = _pad_edges(edge_index_ut)
    tu_s, tu_d = _pad_edges(edge_index_tu)
    zrows = jnp.zeros((N_PAD, D), jnp.float32)
    ones_blk = jnp.ones((G, D), jnp.float32)
    xpad = jnp.zeros((N_PAD - N_NODE, D), jnp.float32)
    xu_p = jnp.concatenate([x_user, xpad])
    xi_p = jnp.concatenate([x_item, xpad])

    s_ut, s_tu, deg_ut, deg_tu = _agg_l1(
        ut_s, ut_d, tu_s, tu_d, xu_p, xi_p, zrows, ones_blk)
    h_item, h_user = _stage1(
        s_ut, deg_ut, xi_p, W1l_ut, b1l_ut.reshape(1, D), W1r_ut,
        s_tu, deg_tu, xu_p, W1l_tu, b1l_tu.reshape(1, D), W1r_tu)

    s2_ut, s2_tu = _agg_l2(
        ut_s, ut_d, tu_s, tu_d, h_user, h_item, zrows, ones_blk)
    z_item, z_user = _stage2(
        s2_ut, deg_ut, h_item, W2l_ut, b2l_ut.reshape(1, D), W2r_ut,
        s2_tu, deg_tu, h_user, W2l_tu, b2l_tu.reshape(1, D), W2r_tu)

    pad = L_PAD - L
    eu1 = jnp.concatenate([edge_label_index[0],
                           jnp.zeros((pad,), edge_label_index.dtype)])
    ei1 = jnp.concatenate([edge_label_index[1],
                           jnp.zeros((pad,), edge_label_index.dtype)])
    zu_g, zi_g = _dec_gather_k(eu1, ei1, z_user, z_item)

    out = _dec_mlp_k(zu_g, zi_g, Wd1[:D], Wd1[D:], bd1.reshape(1, D),
                     Wd2.reshape(1, D), bd2.reshape(1, 1))
    return out[:L]


# speedup vs baseline: 1.3626x; 1.0099x over previous
import functools

import jax
import jax.numpy as jnp
from jax import lax
from jax.experimental import pallas as pl
from jax.experimental.pallas import tpu as pltpu
from jax.experimental.pallas import tpu_sc as plsc

N_NODE = 10000
N_PAD = 10240
E = 320000
L = 100000
D = 128
NS = 16
G = 128

E_CHUNKS_PER_TEC = -(-E // (NS * G))
E_PAD = NS * G * E_CHUNKS_PER_TEC
L_GRP = 7
L_CHUNKS_PER_TEC = -(-L // (NS * G))
L_PAD = NS * G * L_CHUNKS_PER_TEC
L_GROUPS = L_CHUNKS_PER_TEC // L_GRP

_mesh = lambda: plsc.VectorSubcoreMesh(core_axis_name="c", subcore_axis_name="s")


def _make_agg(want_deg):
    scratch = [
        pltpu.VMEM((2 * G,), jnp.int32),
        pltpu.VMEM((G,), jnp.int32),
        pltpu.VMEM((G,), jnp.int32),
        pltpu.VMEM((G, D), jnp.float32),
        pltpu.VMEM((G, D), jnp.float32),
        pltpu.VMEM_SHARED((N_PAD, D), jnp.float32),
        pltpu.SemaphoreType.DMA,
        pltpu.SemaphoreType.DMA,
    ]
    n_out = 4 if want_deg else 2
    out_type = [jax.ShapeDtypeStruct((N_PAD, D), jnp.float32)] * n_out

    @functools.partial(pl.kernel, mesh=_mesh(), out_type=out_type,
                       scratch_types=scratch)
    def agg(ut_src, ut_dst, tu_src, tu_dst, tab_ut, tab_tu, zrows, ones,
            *rest):
        outs = rest[:n_out]
        (sidx2, didx, didxb, rows, rowsb, accum_sh,
         gsem, gsemb) = rest[n_out:]
        c = lax.axis_index("c")
        s = lax.axis_index("s")

        rpt = N_PAD // NS
        sl = pl.ds(s * rpt, rpt)
        pltpu.sync_copy(zrows.at[sl], accum_sh.at[sl])
        plsc.subcore_barrier()

        def run(src1, dst1, tab):
            base = s * E_CHUNKS_PER_TEC * G

            def pairbody(j2, carry):
                off = base + 2 * j2 * G
                pltpu.sync_copy(src1.at[pl.ds(off, 2 * G)], sidx2)
                pltpu.sync_copy(dst1.at[pl.ds(off, G)], didx)
                pltpu.sync_copy(dst1.at[pl.ds(off + G, G)], didxb)
                g0 = pltpu.async_copy(tab.at[sidx2.at[pl.ds(0, G)]],
                                      rows, gsem)
                g1 = pltpu.async_copy(tab.at[sidx2.at[pl.ds(G, G)]],
                                      rowsb, gsemb)
                g0.wait()
                s0 = pltpu.async_copy(rows, accum_sh.at[didx], gsem,
                                      add=True)
                g1.wait()
                s1 = pltpu.async_copy(rowsb, accum_sh.at[didxb], gsemb,
                                      add=True)
                s0.wait()
                s1.wait()
                return carry

            lax.fori_loop(0, E_CHUNKS_PER_TEC // 2, pairbody, 0,
                          unroll=False)
            off = base + (E_CHUNKS_PER_TEC - 1) * G
            pltpu.sync_copy(src1.at[pl.ds(off, G)],
                            sidx2.at[pl.ds(0, G)])
            pltpu.sync_copy(dst1.at[pl.ds(off, G)], didx)
            pltpu.async_copy(tab.at[sidx2.at[pl.ds(0, G)]],
                             rows, gsem).wait()
            pltpu.sync_copy(rows, accum_sh.at[didx], add=True)

        @pl.when(c == 0)
        def _():
            run(ut_src, ut_dst, tab_ut)

        @pl.when(c == 1)
        def _():
            run(tu_src, tu_dst, tab_tu)

        plsc.subcore_barrier()

        @pl.when(c == 0)
        def _():
            pltpu.sync_copy(accum_sh.at[sl], outs[0].at[sl])

        @pl.when(c == 1)
        def _():
            pltpu.sync_copy(accum_sh.at[sl], outs[1].at[sl])

        if want_deg:
            plsc.subcore_barrier()
            pltpu.sync_copy(zrows.at[sl], accum_sh.at[sl])
            pltpu.sync_copy(ones, rows)
            plsc.subcore_barrier()

            def drun(dst1):
                base = s * E_CHUNKS_PER_TEC * G

                def dpair(j2, carry):
                    off = base + 2 * j2 * G
                    pltpu.sync_copy(dst1.at[pl.ds(off, G)], didx)
                    pltpu.sync_copy(dst1.at[pl.ds(off + G, G)], didxb)
                    s0 = pltpu.async_copy(rows, accum_sh.at[didx], gsem,
                                          add=True)
                    s1 = pltpu.async_copy(rows, accum_sh.at[didxb], gsemb,
                                          add=True)
                    s0.wait()
                    s1.wait()
                    return carry

                lax.fori_loop(0, E_CHUNKS_PER_TEC // 2, dpair, 0,
                              unroll=False)
                off = base + (E_CHUNKS_PER_TEC - 1) * G
                pltpu.sync_copy(dst1.at[pl.ds(off, G)], didx)
                pltpu.sync_copy(rows, accum_sh.at[didx], add=True)

            @pl.when(c == 0)
            def _():
                drun(ut_dst)

            @pl.when(c == 1)
            def _():
                drun(tu_dst)

            plsc.subcore_barrier()

            @pl.when(c == 0)
            def _():
                pltpu.sync_copy(accum_sh.at[sl], outs[2].at[sl])

            @pl.when(c == 1)
            def _():
                pltpu.sync_copy(accum_sh.at[sl], outs[3].at[sl])

    return agg


_agg_l1 = _make_agg(True)
_agg_l2 = _make_agg(False)


def _dec_gather():
    out_type = [jax.ShapeDtypeStruct((L_PAD, D), jnp.float32),
                jax.ShapeDtypeStruct((L_PAD, D), jnp.float32)]
    scratch = [
        pltpu.VMEM((L_GRP * G,), jnp.int32),
        pltpu.VMEM((G, D), jnp.float32),
        pltpu.VMEM((G, D), jnp.float32),
        pltpu.SemaphoreType.DMA,
        pltpu.SemaphoreType.DMA,
    ]

    @functools.partial(pl.kernel, mesh=_mesh(), out_type=out_type,
                       scratch_types=scratch)
    def gat(idx_u, idx_i, z_user, z_item, out_u, out_i,
            sidx_blk, rows0, rows1, gs0, gs1):
        rows = [rows0, rows1]
        gsem = [gs0, gs1]
        c = lax.axis_index("c")
        s = lax.axis_index("s")
        base = s * L_CHUNKS_PER_TEC

        def run(idx1, tab, out):
            def group(g, carry):
                goff = (base + g * L_GRP) * G
                pltpu.sync_copy(idx1.at[pl.ds(goff, L_GRP * G)], sidx_blk)
                gcp = [None] * L_GRP
                gcp[0] = pltpu.async_copy(
                    tab.at[sidx_blk.at[pl.ds(0, G)]], rows[0], gsem[0])
                for k in range(L_GRP):
                    b = k % 2
                    if k + 1 < L_GRP:
                        nb = (k + 1) % 2
                        gcp[k + 1] = pltpu.async_copy(
                            tab.at[sidx_blk.at[pl.ds((k + 1) * G, G)]],
                            rows[nb], gsem[nb])
                    gcp[k].wait()
                    pltpu.sync_copy(rows[b], out.at[pl.ds(goff + k * G, G)])
                return carry
            lax.fori_loop(0, L_GROUPS, group, 0, unroll=False)

        @pl.when(c == 0)
        def _():
            run(idx_u, z_user, out_u)

        @pl.when(c == 1)
        def _():
            run(idx_i, z_item, out_i)

    return gat


_dec_gather_k = _dec_gather()

BLK = 1024


def _make_stage(relu):
    def body(s_ut, dg_ut, x_i, Wl_ut, bl_ut, Wr_ut,
             s_tu, dg_tu, x_u, Wl_tu, bl_tu, Wr_tu, h_i, h_u):
        def one(sref, dref, xref, Wl, bl, Wr, out):
            deg = dref[...][:, :1]
            mean = sref[...] / jnp.maximum(deg, 1.0)
            r = jnp.dot(mean, Wl[...], preferred_element_type=jnp.float32)
            r = r + bl[...] + jnp.dot(xref[...], Wr[...],
                                      preferred_element_type=jnp.float32)
            out[...] = jnp.maximum(r, 0.0) if relu else r
        one(s_ut, dg_ut, x_i, Wl_ut, bl_ut, Wr_ut, h_i)
        one(s_tu, dg_tu, x_u, Wl_tu, bl_tu, Wr_tu, h_u)

    srow = pl.BlockSpec((BLK, D), lambda i: (i, 0))
    drow = pl.BlockSpec((BLK, D), lambda i: (i, 0))
    xrow = pl.BlockSpec((BLK, D), lambda i: (i, 0))
    wb = pl.BlockSpec((D, D), lambda i: (0, 0))
    bb = pl.BlockSpec((1, D), lambda i: (0, 0))
    return pl.pallas_call(
        body,
        grid=(N_PAD // BLK,),
        in_specs=[srow, drow, xrow, wb, bb, wb, srow, drow, xrow, wb, bb, wb],
        out_specs=[xrow, xrow],
        out_shape=[jax.ShapeDtypeStruct((N_PAD, D), jnp.float32),
                   jax.ShapeDtypeStruct((N_PAD, D), jnp.float32)],
    )


_stage1 = _make_stage(True)
_stage2 = _make_stage(False)

DBLK = 1024


def _dec_mlp():
    def body(zu, zi, Wt, Wb, b1, w2, b2, out):
        a = jnp.dot(zu[...], Wt[...], preferred_element_type=jnp.float32)
        bq = jnp.dot(zi[...], Wb[...], preferred_element_type=jnp.float32)
        c = jnp.dot(zi[...], Wt[...], preferred_element_type=jnp.float32)
        d = jnp.dot(zu[...], Wb[...], preferred_element_type=jnp.float32)
        f = jnp.maximum(a + bq + b1[...], 0.0) + jnp.maximum(c + d + b1[...], 0.0)
        out[...] = 0.5 * jnp.sum(f * w2[...], axis=1) + b2[0, 0]

    row = pl.BlockSpec((DBLK, D), lambda i: (i, 0))
    wb = pl.BlockSpec((D, D), lambda i: (0, 0))
    bb = pl.BlockSpec((1, D), lambda i: (0, 0))
    sb = pl.BlockSpec(memory_space=pltpu.MemorySpace.SMEM)
    return pl.pallas_call(
        body,
        grid=(L_PAD // DBLK,),
        in_specs=[row, row, wb, wb, bb, bb, sb],
        out_specs=pl.BlockSpec((DBLK,), lambda i: (i,)),
        out_shape=jax.ShapeDtypeStruct((L_PAD,), jnp.float32),
    )


_dec_mlp_k = _dec_mlp()


def _pad_edges(ei):
    n = E_PAD - ei.shape[1]
    src = jnp.concatenate([ei[0], jnp.zeros((n,), ei.dtype)])
    dst = jnp.concatenate([ei[1], jnp.full((n,), N_NODE, ei.dtype)])
    return src, dst


def kernel(x_user, x_item, edge_index_ut, edge_index_tu, edge_label_index,
           W1l_ut, b1l_ut, W1r_ut, W1l_tu, b1l_tu, W1r_tu,
           W2l_ut, b2l_ut, W2r_ut, W2l_tu, b2l_tu, W2r_tu,
           Wd1, bd1, Wd2, bd2):
    ut_s, ut_d = _pad_edges(edge_index_ut)
    tu_s, tu_d = _pad_edges(edge_index_tu)
    zrows = jnp.zeros((N_PAD, D), jnp.float32)
    ones_blk = jnp.ones((G, D), jnp.float32)
    xpad = jnp.zeros((N_PAD - N_NODE, D), jnp.float32)
    xu_p = jnp.concatenate([x_user, xpad])
    xi_p = jnp.concatenate([x_item, xpad])

    s_ut, s_tu, deg_ut, deg_tu = _agg_l1(
        ut_s, ut_d, tu_s, tu_d, xu_p, xi_p, zrows, ones_blk)
    h_item, h_user = _stage1(
        s_ut, deg_ut, xi_p, W1l_ut, b1l_ut.reshape(1, D), W1r_ut,
        s_tu, deg_tu, xu_p, W1l_tu, b1l_tu.reshape(1, D), W1r_tu)

    s2_ut, s2_tu = _agg_l2(
        ut_s, ut_d, tu_s, tu_d, h_user, h_item, zrows, ones_blk)
    z_item, z_user = _stage2(
        s2_ut, deg_ut, h_item, W2l_ut, b2l_ut.reshape(1, D), W2r_ut,
        s2_tu, deg_tu, h_user, W2l_tu, b2l_tu.reshape(1, D), W2r_tu)

    pad = L_PAD - L
    eu1 = jnp.concatenate([edge_label_index[0],
                           jnp.zeros((pad,), edge_label_index.dtype)])
    ei1 = jnp.concatenate([edge_label_index[1],
                           jnp.zeros((pad,), edge_label_index.dtype)])
    zu_g, zi_g = _dec_gather_k(eu1, ei1, z_user, z_item)

    out = _dec_mlp_k(zu_g, zi_g, Wd1[:D], Wd1[D:], bd1.reshape(1, D),
                     Wd2.reshape(1, D), bd2.reshape(1, 1))
    return out[:L]
